# baseline (speedup 1.0000x reference)
import functools

import jax
import jax.numpy as jnp
from jax import lax
from jax.experimental import pallas as pl
from jax.experimental.pallas import tpu as pltpu
from jax.experimental.pallas import tpu_sc as plsc

N = 10000
D = 128
H = 128
E = 320000
G = 8

NC = 2
NS = 16
NW = NC * NS
EW = E // NW
CH = 128
NCH = 80
EWP = NCH * CH
NP = 10240
RPT = NP // NS
DST_PAD = N
NBUF = 5

_f32 = jnp.float32
_bf16 = jnp.bfloat16
_f8 = jnp.float8_e4m3fn

_PERM = []
for _g in range(D // 64):
    _PERM += [_g * 64 + 2 * _i for _i in range(32)]
    _PERM += [_g * 64 + 2 * _i + 1 for _i in range(32)]


def _sc_agg_body(with_counts, y_hbm, srcp_hbm, dstp_hbm, zrow_hbm, z16_hbm,
                 o16_hbm, out_acc, out_cnt, src_v, dst_v, rows, sbuf, ones_v,
                 acc_sh, cnt_sh, gsems, ssems):
    c = lax.axis_index("c")
    s = lax.axis_index("s")
    wid = c * NS + s

    pltpu.sync_copy(srcp_hbm.at[wid], src_v)
    pltpu.sync_copy(dstp_hbm.at[wid], dst_v)
    if with_counts:
        pltpu.sync_copy(z16_hbm, cnt_sh.at[pl.ds(s * RPT, RPT)])
        pltpu.sync_copy(o16_hbm, ones_v)

    pltpu.sync_copy(zrow_hbm, acc_sh.at[pl.ds(s * RPT, RPT)])
    plsc.subcore_barrier()

    def fire_gather(j, b):
        pltpu.async_copy(y_hbm.at[src_v.at[j]], rows.at[b], gsems.at[b])

    def drain_gather(b):
        pltpu.make_async_copy(y_hbm.at[src_v.at[0]], rows.at[b],
                              gsems.at[b]).wait()

    def unpack_rows(b):
        @plsc.parallel_loop(0, CH, 1, unroll=8)
        def _(r):
            for g in range(D // 64):
                v = rows[b, r, pl.ds(g * 64, 64)]
                ev, od = plsc.unpack(v, format=plsc.PackFormat.INTERLEAVED,
                                     preferred_element_type=_bf16)
                sbuf[b, r, pl.ds(g * 64, 32)] = ev
                sbuf[b, r, pl.ds(g * 64 + 32, 32)] = od

    def fire_scatter(j, b):
        pltpu.async_copy(sbuf.at[b], acc_sh.at[dst_v.at[j]], ssems.at[b],
                         add=True)
        if with_counts:
            pltpu.async_copy(ones_v, cnt_sh.at[dst_v.at[j]], ssems.at[b],
                             add=True)

    def drain_scatter(b):
        pltpu.make_async_copy(sbuf.at[b], acc_sh.at[dst_v.at[0]],
                              ssems.at[b]).wait()
        if with_counts:
            pltpu.make_async_copy(ones_v, cnt_sh.at[dst_v.at[0]],
                                  ssems.at[b]).wait()

    for b in range(NBUF):
        fire_gather(b, b)

    def ring(i, carry):
        base = i * NBUF
        for b in range(NBUF):
            drain_gather(b)
            unpack_rows(b)
            fire_scatter(base + b, b)
        for b in range(NBUF):
            drain_scatter(b)

            @pl.when(base + b + NBUF < NCH)
            def _():
                fire_gather(base + b + NBUF, b)
        return carry

    lax.fori_loop(0, NCH // NBUF, ring, 0)
    plsc.subcore_barrier()

    pltpu.sync_copy(acc_sh.at[pl.ds(s * RPT, RPT)],
                    out_acc.at[c].at[pl.ds(s * RPT, RPT)])
    if with_counts:
        pltpu.sync_copy(cnt_sh.at[pl.ds(s * RPT, RPT)],
                        out_cnt.at[c].at[pl.ds(s * RPT, RPT)])


def _make_sc_agg(with_counts):
    mesh = plsc.VectorSubcoreMesh(core_axis_name="c", subcore_axis_name="s",
                                  num_cores=NC, num_subcores=NS)
    return pl.kernel(
        functools.partial(_sc_agg_body, with_counts),
        out_type=[
            jax.ShapeDtypeStruct((NC, NP, D), _bf16),
            jax.ShapeDtypeStruct((NC, NP, 16), _bf16),
        ],
        mesh=mesh,
        scratch_types=[
            pltpu.VMEM((NCH, CH), jnp.int32),
            pltpu.VMEM((NCH, CH), jnp.int32),
            pltpu.VMEM((NBUF, CH, D), _f8),
            pltpu.VMEM((NBUF, CH, D), _bf16),
            pltpu.VMEM((CH, 16), _bf16),
            pltpu.VMEM_SHARED((NP, D), _bf16),
            pltpu.VMEM_SHARED((NP, 16), _bf16),
            pltpu.SemaphoreType.DMA((NBUF,)),
            pltpu.SemaphoreType.DMA((NBUF,)),
        ],
        compiler_params=pltpu.CompilerParams(use_tc_tiling_on_sc=False,
                                             needs_layout_passes=False),
    )


def _tc_root_body(x, wroot, b, root_ref):
    root_ref[...] = (jnp.dot(x[...], wroot[...], preferred_element_type=_f32)
                     + b[...])


def _tc_root(x, wroot, b2d):
    blk = 1000
    return pl.pallas_call(
        _tc_root_body,
        grid=(N // blk,),
        in_specs=[
            pl.BlockSpec((blk, D), lambda i: (i, 0)),
            pl.BlockSpec((D, H), lambda i: (0, 0)),
            pl.BlockSpec((1, H), lambda i: (0, 0)),
        ],
        out_specs=[pl.BlockSpec((blk, H), lambda i: (i, 0))],
        out_shape=[jax.ShapeDtypeStruct((N, H), _f32)],
    )(x, wroot, b2d)[0]


def _tc_layer_body(nblocks, emit_h, p0, p1, c0, c1, root, wrel, oh, *refs):
    if emit_h:
        h_ref, hbf_ref, pool_ref, pacc, cacc = refs
    else:
        pool_ref, pacc, cacc = refs
    i = pl.program_id(0)
    agg = p0[0].astype(_f32) + p1[0].astype(_f32)
    cnt = jnp.maximum(c0[0][:, 0:1].astype(_f32) + c1[0][:, 0:1].astype(_f32), 1.0)
    h = jnp.dot(agg, wrel[...], preferred_element_type=_f32) / cnt
    h = jnp.maximum(h + root[...], 0.0)
    if emit_h:
        h_ref[...] = h
        hbf_ref[...] = h.astype(_f8)

    ohb = oh[...]
    pp = lax.dot_general(ohb, h, (((0,), (0,)), ((), ())),
                         preferred_element_type=_f32)
    cc = jnp.broadcast_to(jnp.sum(ohb, axis=0)[:, None], (G, H))

    @pl.when(i == 0)
    def _():
        pacc[...] = pp
        cacc[...] = cc

    @pl.when(i > 0)
    def _():
        pacc[...] = pacc[...] + pp
        cacc[...] = cacc[...] + cc

    @pl.when(i == nblocks - 1)
    def _():
        pool_ref[...] = pacc[...] / jnp.maximum(cacc[...], 1.0)


def _tc_layer(acc, cnt, root, wrel, oh, emit_h):
    blk = 1000
    nblocks = N // blk
    out_specs = [pl.BlockSpec((G, H), lambda i: (0, 0))]
    out_shape = [jax.ShapeDtypeStruct((G, H), _f32)]
    if emit_h:
        out_specs = [
            pl.BlockSpec((blk, H), lambda i: (i, 0)),
            pl.BlockSpec((blk, H), lambda i: (i, 0)),
        ] + out_specs
        out_shape = [
            jax.ShapeDtypeStruct((N, H), _f32),
            jax.ShapeDtypeStruct((N, H), _f8),
        ] + out_shape
    return pl.pallas_call(
        functools.partial(_tc_layer_body, nblocks, emit_h),
        grid=(nblocks,),
        in_specs=[
            pl.BlockSpec((1, blk, D), lambda i: (0, i, 0)),
            pl.BlockSpec((1, blk, D), lambda i: (1, i, 0)),
            pl.BlockSpec((1, blk, 16), lambda i: (0, i, 0)),
            pl.BlockSpec((1, blk, 16), lambda i: (1, i, 0)),
            pl.BlockSpec((blk, H), lambda i: (i, 0)),
            pl.BlockSpec((D, H), lambda i: (0, 0)),
            pl.BlockSpec((blk, G), lambda i: (i, 0)),
        ],
        out_specs=out_specs,
        out_shape=out_shape,
        scratch_shapes=[
            pltpu.VMEM((G, H), _f32),
            pltpu.VMEM((G, H), _f32),
        ],
    )(acc, acc, cnt, cnt, root, wrel, oh)


def kernel(x, edge_index, batch, W1_rel, b1_rel, W1_root, W2_rel, b2_rel,
           W2_root):
    src = edge_index[0].reshape(NW, EW)
    dst = edge_index[1].reshape(NW, EW)
    srcp = jnp.pad(src, ((0, 0), (0, EWP - EW))).reshape(NW, NCH, CH)
    dstp = jnp.pad(dst, ((0, 0), (0, EWP - EW)),
                   constant_values=DST_PAD).reshape(NW, NCH, CH)
    zrow = jnp.zeros((RPT, D), _bf16)
    z16 = jnp.zeros((RPT, 16), _bf16)
    o16 = jnp.ones((CH, 16), _bf16)
    oh = (batch[:, None] == jnp.arange(G, dtype=batch.dtype)[None, :])
    oh = oh.astype(_f32)
    perm = jnp.asarray(_PERM, dtype=jnp.int32)
    x_f8 = x.astype(_f8)
    W1q = W1_rel[perm, :]
    W2q = W2_rel[perm, :]

    root1 = _tc_root(x, W1_root, b1_rel.reshape(1, H))
    acc1, cnt = _make_sc_agg(True)(x_f8, srcp, dstp, zrow, z16, o16)
    h1, h1f8, pool1 = _tc_layer(acc1, cnt, root1, W1q, oh, True)
    root2 = _tc_root(h1, W2_root, b2_rel.reshape(1, H))
    acc2, _ = _make_sc_agg(False)(h1f8, srcp, dstp, zrow, z16, o16)
    (pool2,) = _tc_layer(acc2, cnt, root2, W2q, oh, False)
    return jnp.concatenate([pool1, pool2], axis=-1)


# speedup vs baseline: 1.0204x; 1.0204x over previous
import functools

import jax
import jax.numpy as jnp
from jax import lax
from jax.experimental import pallas as pl
from jax.experimental.pallas import tpu as pltpu
from jax.experimental.pallas import tpu_sc as plsc

N = 10000
D = 128
H = 128
E = 320000
G = 8

NC = 2
NS = 16
NW = NC * NS
EW = E // NW
CH = 128
NCH = 80
EWP = NCH * CH
NP = 10240
RPT = NP // NS
DST_PAD = N
NBUF = 5

_f32 = jnp.float32
_bf16 = jnp.bfloat16
_f8 = jnp.float8_e4m3fn

_PERM = []
for _g in range(D // 64):
    _PERM += [_g * 64 + 2 * _i for _i in range(32)]
    _PERM += [_g * 64 + 2 * _i + 1 for _i in range(32)]


def _sc_agg_body(with_counts, y_hbm, srcp_hbm, dstp_hbm, zrow_hbm, z16_hbm,
                 o16_hbm, out_acc, out_cnt, src_v, dst_v, rows, sbuf, ones_v,
                 acc_sh, cnt_sh, gsems, ssems):
    c = lax.axis_index("c")
    s = lax.axis_index("s")
    wid = c * NS + s

    pltpu.sync_copy(srcp_hbm.at[wid], src_v)
    pltpu.sync_copy(dstp_hbm.at[wid], dst_v)
    if with_counts:
        pltpu.sync_copy(z16_hbm, cnt_sh.at[pl.ds(s * RPT, RPT)])
        pltpu.sync_copy(o16_hbm, ones_v)

    pltpu.sync_copy(zrow_hbm, acc_sh.at[pl.ds(s * RPT, RPT)])
    plsc.subcore_barrier()

    def fire_gather(j, b):
        pltpu.async_copy(y_hbm.at[src_v.at[j]], rows.at[b], gsems.at[b])

    def drain_gather(b):
        pltpu.make_async_copy(y_hbm.at[src_v.at[0]], rows.at[b],
                              gsems.at[b]).wait()

    def unpack_rows(b):
        @plsc.parallel_loop(0, CH, 1, unroll=8)
        def _(r):
            for g in range(D // 64):
                v = rows[b, r, pl.ds(g * 64, 64)]
                ev, od = plsc.unpack(v, format=plsc.PackFormat.INTERLEAVED,
                                     preferred_element_type=_bf16)
                sbuf[b, r, pl.ds(g * 64, 32)] = ev
                sbuf[b, r, pl.ds(g * 64 + 32, 32)] = od

    def fire_scatter(j, b):
        pltpu.async_copy(sbuf.at[b], acc_sh.at[dst_v.at[j]], ssems.at[b],
                         add=True)
        if with_counts:
            pltpu.async_copy(ones_v, cnt_sh.at[dst_v.at[j]], ssems.at[b],
                             add=True)

    def drain_scatter(b):
        pltpu.make_async_copy(sbuf.at[b], acc_sh.at[dst_v.at[0]],
                              ssems.at[b]).wait()
        if with_counts:
            pltpu.make_async_copy(ones_v, cnt_sh.at[dst_v.at[0]],
                                  ssems.at[b]).wait()

    for b in range(NBUF):
        fire_gather(b, b)

    def ring(i, carry):
        base = i * NBUF
        for b in range(NBUF):
            drain_gather(b)
            unpack_rows(b)
            fire_scatter(base + b, b)
        for b in range(NBUF):
            drain_scatter(b)

            @pl.when(base + b + NBUF < NCH)
            def _():
                fire_gather(base + b + NBUF, b)
        return carry

    lax.fori_loop(0, NCH // NBUF, ring, 0)
    plsc.subcore_barrier()

    pltpu.sync_copy(acc_sh.at[pl.ds(s * RPT, RPT)],
                    out_acc.at[c].at[pl.ds(s * RPT, RPT)])
    if with_counts:
        pltpu.sync_copy(cnt_sh.at[pl.ds(s * RPT, RPT)],
                        out_cnt.at[c].at[pl.ds(s * RPT, RPT)])


def _make_sc_agg(with_counts):
    mesh = plsc.VectorSubcoreMesh(core_axis_name="c", subcore_axis_name="s",
                                  num_cores=NC, num_subcores=NS)
    return pl.kernel(
        functools.partial(_sc_agg_body, with_counts),
        out_type=[
            jax.ShapeDtypeStruct((NC, NP, D), _bf16),
            jax.ShapeDtypeStruct((NC, NP, 16), _bf16),
        ],
        mesh=mesh,
        scratch_types=[
            pltpu.VMEM((NCH, CH), jnp.int32),
            pltpu.VMEM((NCH, CH), jnp.int32),
            pltpu.VMEM((NBUF, CH, D), _f8),
            pltpu.VMEM((NBUF, CH, D), _bf16),
            pltpu.VMEM((CH, 16), _bf16),
            pltpu.VMEM_SHARED((NP, D), _bf16),
            pltpu.VMEM_SHARED((NP, 16), _bf16),
            pltpu.SemaphoreType.DMA((NBUF,)),
            pltpu.SemaphoreType.DMA((NBUF,)),
        ],
        compiler_params=pltpu.CompilerParams(use_tc_tiling_on_sc=False,
                                             needs_layout_passes=False),
    )


def _tc_layer_body(nblocks, p0, p1, c0, c1, x, wrel, wroot, b, oh,
                   h_ref, hbf_ref, pool_ref, pacc, cacc):
    i = pl.program_id(0)
    agg = p0[0].astype(_f32) + p1[0].astype(_f32)
    cnt = jnp.maximum(c0[0][:, 0:1].astype(_f32) + c1[0][:, 0:1].astype(_f32), 1.0)
    h = jnp.dot(agg, wrel[...], preferred_element_type=_f32) / cnt
    h = h + jnp.dot(x[...], wroot[...], preferred_element_type=_f32)
    h = jnp.maximum(h + b[...], 0.0)
    h_ref[...] = h
    hbf_ref[...] = h.astype(_f8)

    ohb = oh[...]
    pp = lax.dot_general(ohb, h, (((0,), (0,)), ((), ())),
                         preferred_element_type=_f32)
    cc = jnp.broadcast_to(jnp.sum(ohb, axis=0)[:, None], (G, H))

    @pl.when(i == 0)
    def _():
        pacc[...] = pp
        cacc[...] = cc

    @pl.when(i > 0)
    def _():
        pacc[...] = pacc[...] + pp
        cacc[...] = cacc[...] + cc

    @pl.when(i == nblocks - 1)
    def _():
        pool_ref[...] = pacc[...] / jnp.maximum(cacc[...], 1.0)


def _tc_layer(acc, cnt, x, wrel, wroot, b2d, oh):
    blk = 1000
    nblocks = N // blk
    return pl.pallas_call(
        functools.partial(_tc_layer_body, nblocks),
        grid=(nblocks,),
        in_specs=[
            pl.BlockSpec((1, blk, D), lambda i: (0, i, 0)),
            pl.BlockSpec((1, blk, D), lambda i: (1, i, 0)),
            pl.BlockSpec((1, blk, 16), lambda i: (0, i, 0)),
            pl.BlockSpec((1, blk, 16), lambda i: (1, i, 0)),
            pl.BlockSpec((blk, D), lambda i: (i, 0)),
            pl.BlockSpec((D, H), lambda i: (0, 0)),
            pl.BlockSpec((D, H), lambda i: (0, 0)),
            pl.BlockSpec((1, H), lambda i: (0, 0)),
            pl.BlockSpec((blk, G), lambda i: (i, 0)),
        ],
        out_specs=[
            pl.BlockSpec((blk, H), lambda i: (i, 0)),
            pl.BlockSpec((blk, H), lambda i: (i, 0)),
            pl.BlockSpec((G, H), lambda i: (0, 0)),
        ],
        out_shape=[
            jax.ShapeDtypeStruct((N, H), _f32),
            jax.ShapeDtypeStruct((N, H), _f8),
            jax.ShapeDtypeStruct((G, H), _f32),
        ],
        scratch_shapes=[
            pltpu.VMEM((G, H), _f32),
            pltpu.VMEM((G, H), _f32),
        ],
    )(acc, acc, cnt, cnt, x, wrel, wroot, b2d, oh)


def kernel(x, edge_index, batch, W1_rel, b1_rel, W1_root, W2_rel, b2_rel,
           W2_root):
    src = edge_index[0].reshape(NW, EW)
    dst = edge_index[1].reshape(NW, EW)
    srcp = jnp.pad(src, ((0, 0), (0, EWP - EW))).reshape(NW, NCH, CH)
    dstp = jnp.pad(dst, ((0, 0), (0, EWP - EW)),
                   constant_values=DST_PAD).reshape(NW, NCH, CH)
    zrow = jnp.zeros((RPT, D), _bf16)
    z16 = jnp.zeros((RPT, 16), _bf16)
    o16 = jnp.ones((CH, 16), _bf16)
    oh = (batch[:, None] == jnp.arange(G, dtype=batch.dtype)[None, :])
    oh = oh.astype(_f32)
    perm = jnp.asarray(_PERM, dtype=jnp.int32)
    x_f8 = x.astype(_f8)
    W1q = W1_rel[perm, :]
    W2q = W2_rel[perm, :]

    acc1, cnt = _make_sc_agg(True)(x_f8, srcp, dstp, zrow, z16, o16)
    h1, h1f8, pool1 = _tc_layer(acc1, cnt, x,
                                W1q, W1_root, b1_rel.reshape(1, H), oh)
    acc2, _ = _make_sc_agg(False)(h1f8, srcp, dstp, zrow, z16, o16)
    _, _, pool2 = _tc_layer(acc2, cnt, h1,
                            W2q, W2_root, b2_rel.reshape(1, H), oh)
    return jnp.concatenate([pool1, pool2], axis=-1)
